# baseline (speedup 1.0000x reference)
import functools

import jax
import jax.numpy as jnp
from jax import lax
from jax.experimental import pallas as pl
from jax.experimental.pallas import tpu as pltpu
from jax.experimental.pallas import tpu_sc as plsc

NUM_DISCRETE = 128
PAD = -1

B = 8
NV = 65536
NF = 131072
NIDX = NF * 3
NC, NS = 2, 16
NW = NC * NS


NBINS = 256
TPB = 4
CH = NV // TPB
SUB = 4096
NSUB = CH // SUB
NBSC = B // NC
SPN = NBSC * NV


def _digit(kv, shift):
    return lax.shift_right_logical(kv, shift) & (NBINS - 1)


def _sort_body(ykey, zkey, pay, out,
               kbuf, zbuf, pbuf, dbuf, cnt, cex, tv,
               sy0, sy1, sz0, sz1, sp0, sp1, shist,
               sem_a, sem_b, sem_c):
    c = lax.axis_index("c")
    s = lax.axis_index("s")
    slot = s // TPB
    m = s % TPB
    b = c * NBSC + slot
    hbm_off = b * NV + m * CH
    spm_off = slot * NV + m * CH
    sbase = slot * (TPB * NBINS) + m * NBINS

    def zero_cnt(_=None):
        def zstep(i, _):
            cnt[pl.ds(i * 16, 16)] = jnp.zeros((16,), jnp.int32)
            return 0
        lax.fori_loop(0, NBINS // 16, zstep, 0)

    def hist_subchunk(shift):
        ones = jnp.ones((16,), jnp.int32)

        def hstep(st, _):
            kv = kbuf[pl.ds(st * 16, 16)]
            d = _digit(kv, shift)
            plsc.addupdate_scatter(cnt, [d], ones)
            return 0
        lax.fori_loop(0, SUB // 16, hstep, 0)

    def dest_subchunk(shift):
        def dstep(st, _):
            kv = kbuf[pl.ds(st * 16, 16)]
            d = _digit(kv, shift)
            base = plsc.load_gather(cnt, [d])
            incl, _unused = plsc.scan_count(d)
            dest = base + incl - 1
            plsc.store_scatter(cnt, [d], dest + 1)
            dbuf[pl.ds(st * 16, 16)] = dest
            return 0
        lax.fori_loop(0, SUB // 16, dstep, 0)

    def compute_bases():
        pltpu.sync_copy(cnt, shist.at[pl.ds(sbase, NBINS)])
        plsc.subcore_barrier()
        pltpu.sync_copy(shist.at[pl.ds(slot * (TPB * NBINS), TPB * NBINS)], cex)

        def bstep(i, carry):
            tot = jnp.zeros((16,), jnp.int32)
            for mm in range(TPB):
                tot = tot + cex[pl.ds(mm * NBINS + i * 16, 16)]
            cs = plsc.cumsum(tot)
            excl = (cs - tot) + carry
            pre = jnp.zeros((16,), jnp.int32)
            for mm in range(TPB):
                chunk = cex[pl.ds(mm * NBINS + i * 16, 16)]
                pre = pre + jnp.where(jnp.full((16,), mm, jnp.int32)
                                      < jnp.full((16,), 1, jnp.int32) * m,
                                      chunk, jnp.zeros((16,), jnp.int32))
            cnt[pl.ds(i * 16, 16)] = excl + pre + slot * NV
            return carry + jnp.sum(tot)
        lax.fori_loop(0, NBINS // 16, bstep, jnp.int32(0))

    def run_pass(shift, src_refs, dst_refs, src_is_hbm):
        off = hbm_off if src_is_hbm else spm_off
        zero_cnt()
        for j in range(NSUB):
            pltpu.sync_copy(src_refs[0].at[pl.ds(off + j * SUB, SUB)], kbuf)
            hist_subchunk(shift)
        compute_bases()
        bufs = (kbuf, zbuf, pbuf)
        for j in range(NSUB):
            pltpu.sync_copy(src_refs[0].at[pl.ds(off + j * SUB, SUB)], kbuf)
            dest_subchunk(shift)
            for a in range(1, len(src_refs)):
                pltpu.sync_copy(src_refs[a].at[pl.ds(off + j * SUB, SUB)],
                                bufs[a])
            cps = []
            for a in range(len(src_refs)):
                cps.append(pltpu.async_copy(bufs[a], dst_refs[a].at[dbuf],
                                            (sem_a, sem_b, sem_c)[a]))
            for cp in cps:
                cp.wait()
        plsc.subcore_barrier()

    run_pass(0, (ykey, zkey, pay), (sy1, sz1, sp1), True)
    run_pass(8, (sy1, sz1, sp1), (sy0, sz0, sp0), False)
    run_pass(16, (sy0, sz0, sp0), (sy1, sz1, sp1), False)
    run_pass(24, (sy1, sz1, sp1), (sy0, sz0, sp0), False)
    run_pass(0, (sz0, sp0), (sz1, sp1), False)
    run_pass(8, (sz1, sp1), (sz0, sp0), False)
    run_pass(16, (sz0, sp0), (sz1, sp1), False)
    run_pass(24, (sz1, sp1), (sz0, sp0), False)
    pltpu.sync_copy(sp0.at[pl.ds(spm_off, CH)], out.at[pl.ds(hbm_off, CH)])


@jax.jit
def _sort_call(ykey, zkey, pay):
    mesh = plsc.VectorSubcoreMesh(
        core_axis_name="c", subcore_axis_name="s", num_cores=NC,
        num_subcores=NS)
    return pl.kernel(
        _sort_body,
        out_type=jax.ShapeDtypeStruct((B * NV,), jnp.int32),
        mesh=mesh,
        scratch_types=[
            pltpu.VMEM((SUB,), jnp.int32),
            pltpu.VMEM((SUB,), jnp.int32),
            pltpu.VMEM((SUB,), jnp.int32),
            pltpu.VMEM((SUB,), jnp.int32),
            pltpu.VMEM((NBINS,), jnp.int32),
            pltpu.VMEM((TPB * NBINS,), jnp.int32),
            pltpu.VMEM((NBINS,), jnp.int32),
            pltpu.VMEM_SHARED((SPN,), jnp.int32),
            pltpu.VMEM_SHARED((SPN,), jnp.int32),
            pltpu.VMEM_SHARED((SPN,), jnp.int32),
            pltpu.VMEM_SHARED((SPN,), jnp.int32),
            pltpu.VMEM_SHARED((SPN,), jnp.int32),
            pltpu.VMEM_SHARED((SPN,), jnp.int32),
            pltpu.VMEM_SHARED((NBSC * TPB * NBINS,), jnp.int32),
            pltpu.SemaphoreType.DMA,
            pltpu.SemaphoreType.DMA,
            pltpu.SemaphoreType.DMA,
        ],
        compiler_params=pltpu.CompilerParams(use_tc_tiling_on_sc=True,
                                             needs_layout_passes=False),
    )(ykey, zkey, pay)


TABW = NBSC * NV * 3
STG = TABW // NS
SCW = NBSC * NIDX * 3
TILEW = SCW // NS
WIN = 6144
NWIN = TILEW // WIN


def _gather_body(tcodes, trecon, fidx3, out_codes, out_recon,
                 idx_v, codes_v, recon_v, scodes, srecon, sem_c, sem_r):
    c = lax.axis_index("c")
    s = lax.axis_index("s")
    pltpu.sync_copy(tcodes.at[pl.ds(c * TABW + s * STG, STG)],
                    scodes.at[pl.ds(s * STG, STG)])
    pltpu.sync_copy(trecon.at[pl.ds(c * TABW + s * STG, STG)],
                    srecon.at[pl.ds(s * STG, STG)])
    plsc.subcore_barrier()
    base = c * SCW + s * TILEW

    def step(w, _):
        off = base + w * WIN
        pltpu.sync_copy(fidx3.at[pl.ds(off, WIN)], idx_v)
        cg = pltpu.async_copy(scodes.at[plsc.Indices(idx_v)], codes_v, sem_c)
        rg = pltpu.async_copy(srecon.at[plsc.Indices(idx_v)], recon_v, sem_r)
        cg.wait()
        pltpu.sync_copy(codes_v, out_codes.at[pl.ds(off, WIN)])
        rg.wait()
        pltpu.sync_copy(recon_v, out_recon.at[pl.ds(off, WIN)])
        return 0

    lax.fori_loop(0, NWIN, step, 0)


@jax.jit
def _gather_call(tcodes, trecon, fidx3):
    mesh = plsc.VectorSubcoreMesh(
        core_axis_name="c", subcore_axis_name="s", num_cores=NC,
        num_subcores=NS)
    return pl.kernel(
        _gather_body,
        out_type=(
            jax.ShapeDtypeStruct((B * NIDX * 3,), jnp.int32),
            jax.ShapeDtypeStruct((B * NIDX * 3,), jnp.float32),
        ),
        mesh=mesh,
        scratch_types=[
            pltpu.VMEM((WIN,), jnp.int32),
            pltpu.VMEM((WIN,), jnp.int32),
            pltpu.VMEM((WIN,), jnp.float32),
            pltpu.VMEM_SHARED((TABW,), jnp.int32),
            pltpu.VMEM_SHARED((TABW,), jnp.float32),
            pltpu.SemaphoreType.DMA,
            pltpu.SemaphoreType.DMA,
        ],
        compiler_params=pltpu.CompilerParams(use_tc_tiling_on_sc=True,
                                             needs_layout_passes=False),
    )(tcodes, trecon, fidx3)


def kernel(vertices, faces):
    min_c = vertices.min(axis=0)
    max_c = vertices.max(axis=0)
    center = (min_c + max_c) / 2
    longest = (max_c - min_c).max()
    v = (vertices - center) / longest

    t = (v - (-1.0)) / 2.0
    t = t * NUM_DISCRETE
    t = t - 0.5
    dcodes = jnp.clip(jnp.round(t).astype(jnp.int32), 0, NUM_DISCRETE - 1)
    packed = (dcodes[..., 0] | (dcodes[..., 1] << 8)
              | (dcodes[..., 2] << 16)).reshape(B * NV)

    bits = lax.bitcast_convert_type(v, jnp.int32)
    skey = bits ^ (jnp.where(bits < 0, jnp.int32(-1), jnp.int32(0))
                   | jnp.int32(-2147483648))
    ykey = skey[..., 1].reshape(B * NV)
    zkey = skey[..., 2].reshape(B * NV)

    sorted_pay = _sort_call(ykey, zkey, packed)

    sc0 = sorted_pay & 255
    sc1 = (sorted_pay >> 8) & 255
    sc2 = (sorted_pay >> 16) & 255
    tcodesf = jnp.stack((sc0, sc1, sc2), axis=-1).reshape(B * NV * 3)
    treconf = (tcodesf.astype(jnp.float32) + 0.5) / NUM_DISCRETE * 2.0 - 1.0

    local_b = (jnp.arange(B, dtype=jnp.int32) % NBSC) * NV
    fidx3 = ((faces.reshape(B, NIDX) + local_b[:, None]) * 3)[..., None] \
        + jnp.arange(3, dtype=jnp.int32)
    fidx3 = fidx3.reshape(B * NIDX * 3)

    codes_flat, recon_flat = _gather_call(tcodesf, treconf, fidx3)

    codes = codes_flat.reshape(B, NF, 3, 3)
    recon = recon_flat.reshape(B, NF, 3, 3)

    flat = codes_flat.reshape(B, NIDX * 3)
    pad = jnp.full((B, 1), PAD, jnp.int32)
    input_ids = jnp.concatenate((pad, flat, pad), axis=1)
    ones = jnp.ones((B, NIDX * 3), jnp.float32)
    attention_mask = jnp.concatenate(
        (pad.astype(jnp.float32), ones, pad.astype(jnp.float32)), axis=1)
    return input_ids, attention_mask, codes, codes, recon


# speedup vs baseline: 1.7301x; 1.7301x over previous
import functools

import jax
import jax.numpy as jnp
from jax import lax
from jax.experimental import pallas as pl
from jax.experimental.pallas import tpu as pltpu
from jax.experimental.pallas import tpu_sc as plsc

NUM_DISCRETE = 128
PAD = -1

B = 8
NV = 65536
NF = 131072
NIDX = NF * 3
NC, NS = 2, 16
NW = NC * NS


NBINS = 256
TPB = 4
CH = NV // TPB
SUB = 4096
NSUB = CH // SUB
NBSC = B // NC
SPN = NBSC * NV


def _digit(kv, shift):
    return lax.shift_right_logical(kv, shift) & (NBINS - 1)


def _sort_body(ykey, zkey, pay, out,
               kbuf, zbuf, pbuf, dbuf, cnt, cex, tv,
               sy0, sy1, sz0, sz1, sp0, sp1, shist,
               sem_a, sem_b, sem_c):
    c = lax.axis_index("c")
    s = lax.axis_index("s")
    slot = s // TPB
    m = s % TPB
    b = c * NBSC + slot
    hbm_off = b * NV + m * CH
    spm_off = slot * NV + m * CH
    sbase = slot * (TPB * NBINS) + m * NBINS

    def zero_cnt(_=None):
        def zstep(i, _):
            cnt[pl.ds(i * 16, 16)] = jnp.zeros((16,), jnp.int32)
            return 0
        lax.fori_loop(0, NBINS // 16, zstep, 0)

    def hist_subchunk(shift):
        ones = jnp.ones((16,), jnp.int32)

        def hstep(st, _):
            kv = kbuf[pl.ds(st * 16, 16)]
            d = _digit(kv, shift)
            plsc.addupdate_scatter(cnt, [d], ones)
            return 0
        lax.fori_loop(0, SUB // 16, hstep, 0)

    def dest_subchunk(shift):
        def dstep(st, _):
            kv = kbuf[pl.ds(st * 16, 16)]
            d = _digit(kv, shift)
            base = plsc.load_gather(cnt, [d])
            incl, _unused = plsc.scan_count(d)
            dest = base + incl - 1
            plsc.store_scatter(cnt, [d], dest + 1)
            dbuf[pl.ds(st * 16, 16)] = dest
            return 0
        lax.fori_loop(0, SUB // 16, dstep, 0)

    def compute_bases():
        pltpu.sync_copy(cnt, shist.at[pl.ds(sbase, NBINS)])
        plsc.subcore_barrier()
        pltpu.sync_copy(shist.at[pl.ds(slot * (TPB * NBINS), TPB * NBINS)], cex)

        def bstep(i, carry):
            tot = jnp.zeros((16,), jnp.int32)
            for mm in range(TPB):
                tot = tot + cex[pl.ds(mm * NBINS + i * 16, 16)]
            cs = plsc.cumsum(tot)
            excl = (cs - tot) + carry
            pre = jnp.zeros((16,), jnp.int32)
            for mm in range(TPB):
                chunk = cex[pl.ds(mm * NBINS + i * 16, 16)]
                pre = pre + jnp.where(jnp.full((16,), mm, jnp.int32)
                                      < jnp.full((16,), 1, jnp.int32) * m,
                                      chunk, jnp.zeros((16,), jnp.int32))
            cnt[pl.ds(i * 16, 16)] = excl + pre + slot * NV
            return carry + jnp.sum(tot)
        lax.fori_loop(0, NBINS // 16, bstep, jnp.int32(0))

    def run_pass(shift, src_refs, dst_refs, src_is_hbm):
        off = hbm_off if src_is_hbm else spm_off
        zero_cnt()
        for j in range(NSUB):
            pltpu.sync_copy(src_refs[0].at[pl.ds(off + j * SUB, SUB)], kbuf)
            hist_subchunk(shift)
        compute_bases()
        bufs = (kbuf, zbuf, pbuf)
        for j in range(NSUB):
            pltpu.sync_copy(src_refs[0].at[pl.ds(off + j * SUB, SUB)], kbuf)
            dest_subchunk(shift)
            for a in range(1, len(src_refs)):
                pltpu.sync_copy(src_refs[a].at[pl.ds(off + j * SUB, SUB)],
                                bufs[a])
            cps = []
            for a in range(len(src_refs)):
                cps.append(pltpu.async_copy(bufs[a], dst_refs[a].at[dbuf],
                                            (sem_a, sem_b, sem_c)[a]))
            for cp in cps:
                cp.wait()
        plsc.subcore_barrier()

    run_pass(0, (ykey, zkey, pay), (sy1, sz1, sp1), True)
    run_pass(8, (sy1, sz1, sp1), (sy0, sz0, sp0), False)
    run_pass(16, (sy0, sz0, sp0), (sy1, sz1, sp1), False)
    run_pass(24, (sy1, sz1, sp1), (sy0, sz0, sp0), False)
    run_pass(0, (sz0, sp0), (sz1, sp1), False)
    run_pass(8, (sz1, sp1), (sz0, sp0), False)
    run_pass(16, (sz0, sp0), (sz1, sp1), False)
    run_pass(24, (sz1, sp1), (sz0, sp0), False)
    pltpu.sync_copy(sp0.at[pl.ds(spm_off, CH)], out.at[pl.ds(hbm_off, CH)])


@jax.jit
def _sort_call(ykey, zkey, pay):
    mesh = plsc.VectorSubcoreMesh(
        core_axis_name="c", subcore_axis_name="s", num_cores=NC,
        num_subcores=NS)
    return pl.kernel(
        _sort_body,
        out_type=jax.ShapeDtypeStruct((B * NV,), jnp.int32),
        mesh=mesh,
        scratch_types=[
            pltpu.VMEM((SUB,), jnp.int32),
            pltpu.VMEM((SUB,), jnp.int32),
            pltpu.VMEM((SUB,), jnp.int32),
            pltpu.VMEM((SUB,), jnp.int32),
            pltpu.VMEM((NBINS,), jnp.int32),
            pltpu.VMEM((TPB * NBINS,), jnp.int32),
            pltpu.VMEM((NBINS,), jnp.int32),
            pltpu.VMEM_SHARED((SPN,), jnp.int32),
            pltpu.VMEM_SHARED((SPN,), jnp.int32),
            pltpu.VMEM_SHARED((SPN,), jnp.int32),
            pltpu.VMEM_SHARED((SPN,), jnp.int32),
            pltpu.VMEM_SHARED((SPN,), jnp.int32),
            pltpu.VMEM_SHARED((SPN,), jnp.int32),
            pltpu.VMEM_SHARED((NBSC * TPB * NBINS,), jnp.int32),
            pltpu.SemaphoreType.DMA,
            pltpu.SemaphoreType.DMA,
            pltpu.SemaphoreType.DMA,
        ],
        compiler_params=pltpu.CompilerParams(use_tc_tiling_on_sc=True,
                                             needs_layout_passes=False),
    )(ykey, zkey, pay)


TABW = NBSC * NV * 3
STG = TABW // NS
SCW = NBSC * NIDX * 3
TILEW = SCW // NS
WIN = 8192
NWIN = TILEW // WIN
PLROW = NF
PLWIN = PLROW // WIN
NPLW = 9 * NBSC * PLWIN // NS


def _gather_body(tcodes, trecon, fidx3, fidxp, out_ids, out_cpl, out_rpl,
                 idx_v, codes_v, recon_v, scodes, srecon, sem_c, sem_r):
    c = lax.axis_index("c")
    s = lax.axis_index("s")
    pltpu.sync_copy(tcodes.at[pl.ds(c * TABW + s * STG, STG)],
                    scodes.at[pl.ds(s * STG, STG)])
    pltpu.sync_copy(trecon.at[pl.ds(c * TABW + s * STG, STG)],
                    srecon.at[pl.ds(s * STG, STG)])
    plsc.subcore_barrier()

    base = c * SCW + s * TILEW

    def step(w, _):
        off = base + w * WIN
        pltpu.sync_copy(fidx3.at[pl.ds(off, WIN)], idx_v)
        cg = pltpu.async_copy(scodes.at[plsc.Indices(idx_v)], codes_v, sem_c)
        cg.wait()
        pltpu.sync_copy(codes_v, out_ids.at[pl.ds(off, WIN)])
        return 0

    lax.fori_loop(0, NWIN, step, 0)

    def pstep(k, _):
        gid = s * NPLW + k
        row = gid // PLWIN
        win = gid - row * PLWIN
        p = row // NBSC
        bloc = row - p * NBSC
        off = (p * B + c * NBSC + bloc) * PLROW + win * WIN
        pltpu.sync_copy(fidxp.at[pl.ds(off, WIN)], idx_v)
        cg = pltpu.async_copy(scodes.at[plsc.Indices(idx_v)], codes_v, sem_c)
        rg = pltpu.async_copy(srecon.at[plsc.Indices(idx_v)], recon_v, sem_r)
        cg.wait()
        pltpu.sync_copy(codes_v, out_cpl.at[pl.ds(off, WIN)])
        rg.wait()
        pltpu.sync_copy(recon_v, out_rpl.at[pl.ds(off, WIN)])
        return 0

    lax.fori_loop(0, NPLW, pstep, 0)


@jax.jit
def _gather_call(tcodes, trecon, fidx3, fidxp):
    mesh = plsc.VectorSubcoreMesh(
        core_axis_name="c", subcore_axis_name="s", num_cores=NC,
        num_subcores=NS)
    return pl.kernel(
        _gather_body,
        out_type=(
            jax.ShapeDtypeStruct((B * NIDX * 3,), jnp.int32),
            jax.ShapeDtypeStruct((B * NIDX * 3,), jnp.int32),
            jax.ShapeDtypeStruct((B * NIDX * 3,), jnp.float32),
        ),
        mesh=mesh,
        scratch_types=[
            pltpu.VMEM((WIN,), jnp.int32),
            pltpu.VMEM((WIN,), jnp.int32),
            pltpu.VMEM((WIN,), jnp.float32),
            pltpu.VMEM_SHARED((TABW,), jnp.int32),
            pltpu.VMEM_SHARED((TABW,), jnp.float32),
            pltpu.SemaphoreType.DMA,
            pltpu.SemaphoreType.DMA,
        ],
        compiler_params=pltpu.CompilerParams(use_tc_tiling_on_sc=True,
                                             needs_layout_passes=False),
    )(tcodes, trecon, fidx3, fidxp)


def kernel(vertices, faces):
    min_c = vertices.min(axis=0)
    max_c = vertices.max(axis=0)
    center = (min_c + max_c) / 2
    longest = (max_c - min_c).max()
    v = (vertices - center) / longest

    t = (v - (-1.0)) / 2.0
    t = t * NUM_DISCRETE
    t = t - 0.5
    dcodes = jnp.clip(jnp.round(t).astype(jnp.int32), 0, NUM_DISCRETE - 1)
    packed = (dcodes[..., 0] | (dcodes[..., 1] << 8)
              | (dcodes[..., 2] << 16)).reshape(B * NV)

    bits = lax.bitcast_convert_type(v, jnp.int32)
    skey = bits ^ (jnp.where(bits < 0, jnp.int32(-1), jnp.int32(0))
                   | jnp.int32(-2147483648))
    ykey = skey[..., 1].reshape(B * NV)
    zkey = skey[..., 2].reshape(B * NV)

    sorted_pay = _sort_call(ykey, zkey, packed)

    sc0 = sorted_pay & 255
    sc1 = (sorted_pay >> 8) & 255
    sc2 = (sorted_pay >> 16) & 255
    tcodesf = jnp.stack((sc0, sc1, sc2), axis=-1).reshape(B * NV * 3)
    treconf = (tcodesf.astype(jnp.float32) + 0.5) / NUM_DISCRETE * 2.0 - 1.0

    local_b = (jnp.arange(B, dtype=jnp.int32) % NBSC) * NV
    fa = faces + local_b[:, None, None]
    fidx3 = (fa.reshape(B, NIDX) * 3)[..., None] \
        + jnp.arange(3, dtype=jnp.int32)
    fidx3 = fidx3.reshape(B * NIDX * 3)
    fat = jnp.transpose(fa, (2, 0, 1))
    fidxp = (fat[:, None] * 3
             + jnp.arange(3, dtype=jnp.int32)[None, :, None, None])
    fidxp = fidxp.reshape(B * NIDX * 3)

    ids_body, codes_pl, recon_pl = _gather_call(tcodesf, treconf, fidx3,
                                                fidxp)

    codes = codes_pl.reshape(3, 3, B, NF).transpose(2, 3, 0, 1)
    recon = recon_pl.reshape(3, 3, B, NF).transpose(2, 3, 0, 1)

    flat = ids_body.reshape(B, NIDX * 3)
    pad = jnp.full((B, 1), PAD, jnp.int32)
    input_ids = jnp.concatenate((pad, flat, pad), axis=1)
    ones = jnp.ones((B, NIDX * 3), jnp.float32)
    attention_mask = jnp.concatenate(
        (pad.astype(jnp.float32), ones, pad.astype(jnp.float32)), axis=1)
    return input_ids, attention_mask, codes, codes, recon
